# Initial kernel scaffold; baseline (speedup 1.0000x reference)
import jax
import jax.numpy as jnp
from jax.experimental import pallas as pl


def kernel(x, degree, atom_table, degree_table, graph_token):
    raise NotImplementedError("write your pallas kernel here")


# speedup vs baseline: 9.1394x; 9.1394x over previous
import jax
import jax.numpy as jnp
from jax import lax
from jax.experimental import pallas as pl
from jax.experimental.pallas import tpu as pltpu
from jax.experimental.pallas import tpu_sc as plsc

B, N, F, D = 1024, 128, 9, 64
NC, NS = 2, 16
NW = NC * NS
BPW = B // NW
C = 64
ROWS = C * F


def _sc_body(xr_hbm, deg_hbm, atom_hbm, dtab_hbm, tok_hbm, out_hbm,
             aidx, didx, arows, grows, obuf, sem):
    wid = lax.axis_index("s") * NC + lax.axis_index("c")

    pltpu.sync_copy(tok_hbm, obuf.at[pl.ds(0, 1)])

    def batch_body(i, carry):
        b = wid * BPW + i

        def do_half(h):
            g = b * 2 + h
            pltpu.sync_copy(xr_hbm.at[g], aidx)
            pltpu.sync_copy(deg_hbm.at[g], didx)
            handles = [
                pltpu.async_copy(atom_hbm.at[aidx.at[j]],
                                 arows.at[pl.ds(j * C, C)], sem)
                for j in range(F)
            ]
            handles.append(
                pltpu.async_copy(dtab_hbm.at[didx.at[0]], grows, sem))
            for hh in handles:
                hh.wait()

            def node_body(c, acc_carry):
                r0 = c * F
                for col in range(D // 16):
                    cs = pl.ds(col * 16, 16)
                    acc = grows[c, cs]
                    for j in range(F):
                        acc = acc + arows[r0 + j, cs]
                    obuf[1 + h * C + c, cs] = acc
                return acc_carry

            lax.fori_loop(0, C, node_body, 0)

        do_half(0)
        do_half(1)
        pltpu.sync_copy(obuf, out_hbm.at[b])
        return carry

    lax.fori_loop(0, BPW, batch_body, 0)


def kernel(x, degree, atom_table, degree_table, graph_token):
    xr = x.reshape(B * 2, F, C)
    degf = degree.reshape(B * 2, 1, C)
    mesh = plsc.VectorSubcoreMesh(core_axis_name="c", subcore_axis_name="s")
    run = pl.kernel(
        _sc_body,
        out_type=jax.ShapeDtypeStruct((B, N + 1, D), jnp.float32),
        mesh=mesh,
        scratch_types=[
            pltpu.VMEM((F, C), jnp.int32),
            pltpu.VMEM((1, C), jnp.int32),
            pltpu.VMEM((ROWS, D), jnp.float32),
            pltpu.VMEM((C, D), jnp.float32),
            pltpu.VMEM((N + 1, D), jnp.float32),
            pltpu.SemaphoreType.DMA,
        ],
        compiler_params=pltpu.CompilerParams(use_tc_tiling_on_sc=False),
    )
    return run(xr, degf, atom_table, degree_table, graph_token)


# speedup vs baseline: 12.4951x; 1.3672x over previous
import jax
import jax.numpy as jnp
from jax import lax
from jax.experimental import pallas as pl
from jax.experimental.pallas import tpu as pltpu
from jax.experimental.pallas import tpu_sc as plsc

B, N, F, D = 1024, 128, 9, 64
NC, NS = 2, 16
NW = NC * NS
BPW = B // NW
C = 64
ROWS = C * F


def _sc_body(xr_hbm, deg_hbm, atom_hbm, dtab_hbm, tok_hbm, out_hbm,
             aidx0, aidx1, didx0, didx1, arows0, arows1, grows0, grows1,
             obuf, semI0, semI1, semG0, semG1, semO):
    wid = lax.axis_index("s") * NC + lax.axis_index("c")
    g0 = wid * BPW * 2

    aidx = (aidx0, aidx1)
    didx = (didx0, didx1)
    arows = (arows0, arows1)
    grows = (grows0, grows1)
    semI = (semI0, semI1)
    semG = (semG0, semG1)

    def fire_idx(chunk, h):
        pltpu.async_copy(xr_hbm.at[chunk], aidx[h], semI[h])
        pltpu.async_copy(deg_hbm.at[chunk], didx[h], semI[h])

    def wait_idx(h):
        pltpu.make_async_copy(xr_hbm.at[0], aidx[h], semI[h]).wait()
        pltpu.make_async_copy(deg_hbm.at[0], didx[h], semI[h]).wait()

    def fire_gathers(h):
        for j in range(F):
            pltpu.async_copy(atom_hbm.at[aidx[h].at[j]],
                             arows[h].at[pl.ds(j * C, C)], semG[h])
        pltpu.async_copy(dtab_hbm.at[didx[h].at[0]], grows[h], semG[h])

    def wait_gathers(h):
        for j in range(F):
            pltpu.make_async_copy(atom_hbm.at[pl.ds(0, C)],
                                  arows[h].at[pl.ds(j * C, C)],
                                  semG[h]).wait()
        pltpu.make_async_copy(dtab_hbm.at[pl.ds(0, C)], grows[h],
                              semG[h]).wait()

    def compute(h):
        def node_body(c, acc_carry):
            r0 = c * F
            for col in range(D // 16):
                cs = pl.ds(col * 16, 16)
                acc = grows[h][c, cs]
                for j in range(F):
                    acc = acc + arows[h][r0 + j, cs]
                obuf[1 + h * C + c, cs] = acc
            return acc_carry

        lax.fori_loop(0, C, node_body, 0)

    pltpu.sync_copy(tok_hbm, obuf.at[pl.ds(0, 1)])

    fire_idx(g0, 0)
    fire_idx(g0 + 1, 1)
    wait_idx(0)
    fire_gathers(0)

    def batch_body(i, carry):
        b = wid * BPW + i
        last = i == BPW - 1

        wait_gathers(0)
        wait_idx(1)
        fire_gathers(1)

        @pl.when(i > 0)
        def _():
            pltpu.make_async_copy(obuf, out_hbm.at[0], semO).wait()

        compute(0)

        wait_gathers(1)

        @pl.when(jnp.logical_not(last))
        def _():
            fire_idx(g0 + 2 * i + 2, 0)
            fire_idx(g0 + 2 * i + 3, 1)
            wait_idx(0)
            fire_gathers(0)

        compute(1)
        pltpu.async_copy(obuf, out_hbm.at[b], semO)
        return carry

    lax.fori_loop(0, BPW, batch_body, 0)
    pltpu.make_async_copy(obuf, out_hbm.at[0], semO).wait()


def kernel(x, degree, atom_table, degree_table, graph_token):
    xr = x.reshape(B * 2, F, C)
    degf = degree.reshape(B * 2, 1, C)
    mesh = plsc.VectorSubcoreMesh(core_axis_name="c", subcore_axis_name="s")
    run = pl.kernel(
        _sc_body,
        out_type=jax.ShapeDtypeStruct((B, N + 1, D), jnp.float32),
        mesh=mesh,
        scratch_types=[
            pltpu.VMEM((F, C), jnp.int32),
            pltpu.VMEM((F, C), jnp.int32),
            pltpu.VMEM((1, C), jnp.int32),
            pltpu.VMEM((1, C), jnp.int32),
            pltpu.VMEM((ROWS, D), jnp.float32),
            pltpu.VMEM((ROWS, D), jnp.float32),
            pltpu.VMEM((C, D), jnp.float32),
            pltpu.VMEM((C, D), jnp.float32),
            pltpu.VMEM((N + 1, D), jnp.float32),
            pltpu.SemaphoreType.DMA,
            pltpu.SemaphoreType.DMA,
            pltpu.SemaphoreType.DMA,
            pltpu.SemaphoreType.DMA,
            pltpu.SemaphoreType.DMA,
        ],
        compiler_params=pltpu.CompilerParams(use_tc_tiling_on_sc=False),
    )
    return run(xr, degf, atom_table, degree_table, graph_token)


# speedup vs baseline: 12.6278x; 1.0106x over previous
import jax
import jax.numpy as jnp
from jax import lax
from jax.experimental import pallas as pl
from jax.experimental.pallas import tpu as pltpu
from jax.experimental.pallas import tpu_sc as plsc

B, N, F, D = 1024, 128, 9, 64
NC, NS = 2, 16
NW = NC * NS
BPW = B // NW
C = 64
ROWS = C * F
OG = (N + 1) * D


def _sc_body(x_hbm, deg_hbm, atom_hbm, dtab_hbm, tok_hbm, out_hbm,
             aidx0, aidx1, didx0, didx1, arows0, arows1, grows0, grows1,
             obuf, semI0, semI1, semG0, semG1, semO):
    wid = lax.axis_index("s") * NC + lax.axis_index("c")
    g0 = wid * BPW * 2

    aidx = (aidx0, aidx1)
    didx = (didx0, didx1)
    arows = (arows0, arows1)
    grows = (grows0, grows1)
    semI = (semI0, semI1)
    semG = (semG0, semG1)

    def fire_idx(chunk, h):
        pltpu.async_copy(x_hbm.at[pl.ds(chunk * ROWS, ROWS)], aidx[h],
                         semI[h])
        pltpu.async_copy(deg_hbm.at[pl.ds(chunk * C, C)], didx[h], semI[h])

    def wait_idx(h):
        pltpu.make_async_copy(x_hbm.at[pl.ds(0, ROWS)], aidx[h],
                              semI[h]).wait()
        pltpu.make_async_copy(deg_hbm.at[pl.ds(0, C)], didx[h],
                              semI[h]).wait()

    def fire_gathers(h):
        for j in range(F):
            pltpu.async_copy(atom_hbm.at[aidx[h].at[pl.ds(j * C, C)]],
                             arows[h].at[pl.ds(j * C, C)], semG[h])
        pltpu.async_copy(dtab_hbm.at[didx[h]], grows[h], semG[h])

    def wait_gathers(h):
        for j in range(F):
            pltpu.make_async_copy(atom_hbm.at[pl.ds(0, C)],
                                  arows[h].at[pl.ds(j * C, C)],
                                  semG[h]).wait()
        pltpu.make_async_copy(dtab_hbm.at[pl.ds(0, C)], grows[h],
                              semG[h]).wait()

    def compute(h):
        def node_body(c, acc_carry):
            r0 = c * F
            o0 = (1 + h * C + c) * D
            for col in range(D // 16):
                cs = pl.ds(col * 16, 16)
                acc = grows[h][c, cs]
                for j in range(F):
                    acc = acc + arows[h][r0 + j, cs]
                obuf[pl.ds(o0 + col * 16, 16)] = acc
            return acc_carry

        lax.fori_loop(0, C, node_body, 0)

    pltpu.sync_copy(tok_hbm, obuf.at[pl.ds(0, D)])

    fire_idx(g0, 0)
    fire_idx(g0 + 1, 1)
    wait_idx(0)
    fire_gathers(0)

    def batch_body(i, carry):
        b = wid * BPW + i
        last = i == BPW - 1

        wait_gathers(0)
        wait_idx(1)
        fire_gathers(1)

        @pl.when(i > 0)
        def _():
            pltpu.make_async_copy(obuf, out_hbm.at[pl.ds(0, OG)], semO).wait()

        compute(0)

        wait_gathers(1)

        @pl.when(jnp.logical_not(last))
        def _():
            fire_idx(g0 + 2 * i + 2, 0)
            fire_idx(g0 + 2 * i + 3, 1)
            wait_idx(0)
            fire_gathers(0)

        compute(1)
        pltpu.async_copy(obuf, out_hbm.at[pl.ds(b * OG, OG)], semO)
        return carry

    lax.fori_loop(0, BPW, batch_body, 0)
    pltpu.make_async_copy(obuf, out_hbm.at[pl.ds(0, OG)], semO).wait()


def kernel(x, degree, atom_table, degree_table, graph_token):
    xf = x.reshape(B * N * F)
    degf = degree.reshape(B * N)
    tokf = graph_token.reshape(D)
    mesh = plsc.VectorSubcoreMesh(core_axis_name="c", subcore_axis_name="s")
    run = pl.kernel(
        _sc_body,
        out_type=jax.ShapeDtypeStruct((B * OG,), jnp.float32),
        mesh=mesh,
        scratch_types=[
            pltpu.VMEM((ROWS,), jnp.int32),
            pltpu.VMEM((ROWS,), jnp.int32),
            pltpu.VMEM((C,), jnp.int32),
            pltpu.VMEM((C,), jnp.int32),
            pltpu.VMEM((ROWS, D), jnp.float32),
            pltpu.VMEM((ROWS, D), jnp.float32),
            pltpu.VMEM((C, D), jnp.float32),
            pltpu.VMEM((C, D), jnp.float32),
            pltpu.VMEM(((N + 1) * D,), jnp.float32),
            pltpu.SemaphoreType.DMA,
            pltpu.SemaphoreType.DMA,
            pltpu.SemaphoreType.DMA,
            pltpu.SemaphoreType.DMA,
            pltpu.SemaphoreType.DMA,
        ],
        compiler_params=pltpu.CompilerParams(use_tc_tiling_on_sc=False),
    )
    out = run(xf, degf, atom_table, degree_table, tokf)
    return out.reshape(B, N + 1, D)


# speedup vs baseline: 19.7091x; 1.5608x over previous
import jax
import jax.numpy as jnp
from jax import lax
from jax.experimental import pallas as pl
from jax.experimental.pallas import tpu as pltpu
from jax.experimental.pallas import tpu_sc as plsc

B, N, F, D = 1024, 128, 9, 64
NC, NS = 2, 16
NW = NC * NS
BPW = B // NW
C = 64
ROWS = C * F
OG = (N + 1) * D


def _sc_body(xt_hbm, deg_hbm, atom_hbm, dtab_hbm, tok_hbm, out_hbm,
             aidx, didx0, didx1, arows0, arows1, grows0, grows1,
             obuf, semI0, semI1, semG0, semG1, semO):
    wid = lax.axis_index("s") * NC + lax.axis_index("c")
    b0 = wid * BPW

    didx = (didx0, didx1)
    arows = (arows0, arows1)
    grows = (grows0, grows1)
    semI = (semI0, semI1)
    semG = (semG0, semG1)

    pltpu.sync_copy(xt_hbm.at[:, pl.ds(b0, BPW), :], aidx)
    pltpu.sync_copy(tok_hbm, obuf.at[pl.ds(0, D)])

    def fire_didx(b, h):
        pltpu.async_copy(deg_hbm.at[pl.ds(b * N + h * C, C)], didx[h],
                         semI[h])

    def wait_didx(h):
        pltpu.make_async_copy(deg_hbm.at[pl.ds(0, C)], didx[h],
                              semI[h]).wait()

    def fire_gathers(i, h):
        for j in range(F):
            pltpu.async_copy(atom_hbm.at[aidx.at[j, i, pl.ds(h * C, C)]],
                             arows[h].at[pl.ds(j * C, C)], semG[h])
        pltpu.async_copy(dtab_hbm.at[didx[h]], grows[h], semG[h])

    def wait_gathers(h):
        for j in range(F):
            pltpu.make_async_copy(atom_hbm.at[pl.ds(0, C)],
                                  arows[h].at[pl.ds(j * C, C)],
                                  semG[h]).wait()
        pltpu.make_async_copy(dtab_hbm.at[pl.ds(0, C)], grows[h],
                              semG[h]).wait()

    def compute(h):
        def node_body(c, acc_carry):
            o0 = (1 + h * C + c) * D
            for col in range(D // 16):
                cs = pl.ds(col * 16, 16)
                acc = grows[h][c, cs]
                for j in range(F):
                    acc = acc + arows[h][j * C + c, cs]
                obuf[pl.ds(o0 + col * 16, 16)] = acc
            return acc_carry

        lax.fori_loop(0, C, node_body, 0)

    fire_didx(b0, 0)
    fire_didx(b0, 1)
    wait_didx(0)
    fire_gathers(0, 0)

    def batch_body(i, carry):
        b = b0 + i
        last = i == BPW - 1

        wait_gathers(0)

        @pl.when(jnp.logical_not(last))
        def _():
            fire_didx(b + 1, 0)

        wait_didx(1)
        fire_gathers(i, 1)

        @pl.when(i > 0)
        def _():
            pltpu.make_async_copy(obuf, out_hbm.at[pl.ds(0, OG)], semO).wait()

        compute(0)

        wait_gathers(1)

        @pl.when(jnp.logical_not(last))
        def _():
            fire_didx(b + 1, 1)
            wait_didx(0)
            fire_gathers(i + 1, 0)

        compute(1)
        pltpu.async_copy(obuf, out_hbm.at[pl.ds(b * OG, OG)], semO)
        return carry

    lax.fori_loop(0, BPW, batch_body, 0)
    pltpu.make_async_copy(obuf, out_hbm.at[pl.ds(0, OG)], semO).wait()


def kernel(x, degree, atom_table, degree_table, graph_token):
    xt = jnp.transpose(x, (2, 0, 1))
    degf = degree.reshape(B * N)
    tokf = graph_token.reshape(D)
    mesh = plsc.VectorSubcoreMesh(core_axis_name="c", subcore_axis_name="s")
    run = pl.kernel(
        _sc_body,
        out_type=jax.ShapeDtypeStruct((B * OG,), jnp.float32),
        mesh=mesh,
        scratch_types=[
            pltpu.VMEM((F, BPW, N), jnp.int32),
            pltpu.VMEM((C,), jnp.int32),
            pltpu.VMEM((C,), jnp.int32),
            pltpu.VMEM((ROWS, D), jnp.float32),
            pltpu.VMEM((ROWS, D), jnp.float32),
            pltpu.VMEM((C, D), jnp.float32),
            pltpu.VMEM((C, D), jnp.float32),
            pltpu.VMEM(((N + 1) * D,), jnp.float32),
            pltpu.SemaphoreType.DMA,
            pltpu.SemaphoreType.DMA,
            pltpu.SemaphoreType.DMA,
            pltpu.SemaphoreType.DMA,
            pltpu.SemaphoreType.DMA,
        ],
        compiler_params=pltpu.CompilerParams(use_tc_tiling_on_sc=False),
    )
    out = run(xt, degf, atom_table, degree_table, tokf)
    return out.reshape(B, N + 1, D)
